# baseline (speedup 1.0000x reference)
import functools

import jax
import jax.numpy as jnp
from jax import lax
from jax.experimental import pallas as pl
from jax.experimental.pallas import tpu as pltpu
from jax.experimental.pallas import tpu_sc as plsc

_B = 4
_N = 10000
_E = 160000
_D = 128
_DO = 8
_NT = _B * _N
_TILES = 16
_BS = 128
_NB = 80
_CHUNK = 8
_NCHUNK = _NB // _CHUNK
_NBX = _NB + _CHUNK
_RPT = 624
_REM_OFF = _RPT * _TILES
_REM = _N - _REM_OFF
_NDUMP = 256
_ACC_ROWS = _N + _NDUMP
_EPS = 1e-5


def _make_sc_segsum():
    mesh = plsc.VectorSubcoreMesh(core_axis_name="c", subcore_axis_name="s")

    @functools.partial(
        pl.kernel,
        mesh=mesh,
        out_type=jax.ShapeDtypeStruct((_NT, _D), jnp.float32),
        scratch_types=[
            pltpu.VMEM_SHARED((_ACC_ROWS, _D), jnp.float32),
            pltpu.VMEM((2, _CHUNK, _BS), jnp.int32),
            pltpu.VMEM((_NB, _BS), jnp.int32),
            pltpu.VMEM((2, _BS, _D), jnp.float32),
            pltpu.SemaphoreType.DMA((2,)),
            pltpu.SemaphoreType.DMA((2,)),
            pltpu.SemaphoreType.DMA,
        ],
    )
    def segsum(h_hbm, cols_hbm, rows_hbm, out_hbm, acc, idx_c, idx_r, buf,
               gsem, ssem, isem):
        c = lax.axis_index("c")
        s = lax.axis_index("s")
        pltpu.sync_copy(rows_hbm.at[s], idx_r)
        for gi in range(_B // 2):
            b = c + 2 * gi
            pltpu.sync_copy(
                h_hbm.at[pl.ds(b * _N + s * _RPT, _RPT)],
                acc.at[pl.ds(s * _RPT, _RPT)],
            )

            @pl.when(s == 0)
            def _():
                pltpu.sync_copy(
                    h_hbm.at[pl.ds(b * _N + _REM_OFF, _REM)],
                    acc.at[pl.ds(_REM_OFF, _REM)],
                )

            pltpu.sync_copy(cols_hbm.at[b, s, pl.ds(0, _CHUNK)], idx_c.at[0])
            plsc.subcore_barrier()

            def chunk_body(ci, carry):
                cur = ci % 2
                jb = ci * _CHUNK

                def gath_start(k):
                    for hh in range(2):
                        pltpu.async_copy(
                            h_hbm.at[idx_c.at[cur, k, pl.ds(hh * 64, 64)]],
                            buf.at[k % 2, pl.ds(hh * 64, 64)],
                            gsem.at[k % 2])

                def gath_wait(k):
                    for hh in range(2):
                        pltpu.make_async_copy(
                            h_hbm.at[idx_c.at[cur, k, pl.ds(hh * 64, 64)]],
                            buf.at[k % 2, pl.ds(hh * 64, 64)],
                            gsem.at[k % 2]).wait()

                def scat_start(k):
                    pltpu.async_copy(buf.at[k % 2], acc.at[idx_r.at[jb + k]],
                                     ssem.at[k % 2], add=True)

                def scat_wait(k):
                    pltpu.make_async_copy(buf.at[k % 2],
                                          acc.at[idx_r.at[jb + k]],
                                          ssem.at[k % 2]).wait()

                off = pl.multiple_of((ci + 1) * _CHUNK, 8)
                pltpu.async_copy(cols_hbm.at[b, s, pl.ds(off, _CHUNK)],
                                 idx_c.at[(ci + 1) % 2], isem)

                for k in range(_CHUNK):
                    if k >= 2:
                        scat_wait(k - 2)
                    gath_start(k)
                    if k >= 1:
                        gath_wait(k - 1)
                        scat_start(k - 1)
                gath_wait(_CHUNK - 1)
                scat_start(_CHUNK - 1)
                scat_wait(_CHUNK - 2)
                scat_wait(_CHUNK - 1)
                pltpu.make_async_copy(
                    cols_hbm.at[b, s, pl.ds(off, _CHUNK)],
                    idx_c.at[(ci + 1) % 2], isem).wait()
                return carry

            lax.fori_loop(0, _NCHUNK, chunk_body, 0)
            plsc.subcore_barrier()
            pltpu.sync_copy(
                acc.at[pl.ds(s * _RPT, _RPT)],
                out_hbm.at[pl.ds(b * _N + s * _RPT, _RPT)],
            )

            @pl.when(s == 0)
            def _():
                pltpu.sync_copy(
                    acc.at[pl.ds(_REM_OFF, _REM)],
                    out_hbm.at[pl.ds(b * _N + _REM_OFF, _REM)],
                )

            plsc.subcore_barrier()

    return segsum


_sc_segsum = _make_sc_segsum()


_BLK = 2000
_GRID = _NT // _BLK


def _stats_update(i, st_ref, z):
    ps = jnp.concatenate(
        [jnp.sum(z, axis=0, keepdims=True), jnp.sum(z * z, axis=0, keepdims=True)],
        axis=0,
    )

    @pl.when(i == 0)
    def _():
        st_ref[...] = jnp.zeros_like(st_ref)

    st_ref[...] += ps


def _mm_stats_body(x_ref, w_ref, b_ref, z_ref, st_ref):
    i = pl.program_id(0)
    z = jnp.dot(x_ref[...], w_ref[...], preferred_element_type=jnp.float32) + b_ref[...]
    z_ref[...] = z
    _stats_update(i, st_ref, z)


def _mm_stats(x, w, b):
    return pl.pallas_call(
        _mm_stats_body,
        grid=(_GRID,),
        in_specs=[
            pl.BlockSpec((_BLK, _D), lambda i: (i, 0)),
            pl.BlockSpec((_D, _D), lambda i: (0, 0)),
            pl.BlockSpec((1, _D), lambda i: (0, 0)),
        ],
        out_specs=[
            pl.BlockSpec((_BLK, _D), lambda i: (i, 0)),
            pl.BlockSpec((2, _D), lambda i: (0, 0)),
        ],
        out_shape=[
            jax.ShapeDtypeStruct((_NT, _D), jnp.float32),
            jax.ShapeDtypeStruct((2, _D), jnp.float32),
        ],
    )(x, w, b)


def _affine_from_stats(st, g, be):
    mean = st[0:1, :] * (1.0 / _NT)
    var = st[1:2, :] * (1.0 / _NT) - mean * mean
    scale = g * lax.rsqrt(var + _EPS)
    shift = be - mean * scale
    return scale, shift


def _bn_mm_stats_body(z1_ref, st_ref, g_ref, be_ref, w_ref, b_ref, z2_ref, st2_ref):
    i = pl.program_id(0)
    scale, shift = _affine_from_stats(st_ref[...], g_ref[...], be_ref[...])
    x = jnp.maximum(z1_ref[...] * scale + shift, 0.0)
    z2 = jnp.dot(x, w_ref[...], preferred_element_type=jnp.float32) + b_ref[...]
    z2_ref[...] = z2
    _stats_update(i, st2_ref, z2)


def _bn_mm_stats(z1, st, g, be, w, b):
    return pl.pallas_call(
        _bn_mm_stats_body,
        grid=(_GRID,),
        in_specs=[
            pl.BlockSpec((_BLK, _D), lambda i: (i, 0)),
            pl.BlockSpec((2, _D), lambda i: (0, 0)),
            pl.BlockSpec((1, _D), lambda i: (0, 0)),
            pl.BlockSpec((1, _D), lambda i: (0, 0)),
            pl.BlockSpec((_D, _D), lambda i: (0, 0)),
            pl.BlockSpec((1, _D), lambda i: (0, 0)),
        ],
        out_specs=[
            pl.BlockSpec((_BLK, _D), lambda i: (i, 0)),
            pl.BlockSpec((2, _D), lambda i: (0, 0)),
        ],
        out_shape=[
            jax.ShapeDtypeStruct((_NT, _D), jnp.float32),
            jax.ShapeDtypeStruct((2, _D), jnp.float32),
        ],
    )(z1, st, g, be, w, b)


def _bn_relu_body(z_ref, st_ref, g_ref, be_ref, h_ref):
    scale, shift = _affine_from_stats(st_ref[...], g_ref[...], be_ref[...])
    h_ref[...] = jnp.maximum(z_ref[...] * scale + shift, 0.0)


def _bn_relu(z, st, g, be):
    return pl.pallas_call(
        _bn_relu_body,
        grid=(_GRID,),
        in_specs=[
            pl.BlockSpec((_BLK, _D), lambda i: (i, 0)),
            pl.BlockSpec((2, _D), lambda i: (0, 0)),
            pl.BlockSpec((1, _D), lambda i: (0, 0)),
            pl.BlockSpec((1, _D), lambda i: (0, 0)),
        ],
        out_specs=pl.BlockSpec((_BLK, _D), lambda i: (i, 0)),
        out_shape=jax.ShapeDtypeStruct((_NT, _D), jnp.float32),
    )(z, st, g, be)


def _pred_body(hs_ref, w_ref, bsum_ref, out_ref):
    prod = hs_ref[...] * w_ref[...][:, None, :]
    s = jnp.sum(jnp.sum(prod, axis=0), axis=1, keepdims=True)
    out_ref[...] = jax.nn.sigmoid(s + bsum_ref[...])


def _pred(hs, w, bsum):
    return pl.pallas_call(
        _pred_body,
        in_specs=[
            pl.BlockSpec(hs.shape, lambda: (0, 0, 0)),
            pl.BlockSpec(w.shape, lambda: (0, 0)),
            pl.BlockSpec(bsum.shape, lambda: (0, 0)),
        ],
        out_specs=pl.BlockSpec((_B * _DO, 1), lambda: (0, 0)),
        out_shape=jax.ShapeDtypeStruct((_B * _DO, 1), jnp.float32),
    )(hs, w, bsum)


def kernel(batch_feature, edge_mat, params):
    em = params["emotion_embeddings"]
    bf = batch_feature.at[:, :_DO, :].set(jnp.broadcast_to(em, (_B, _DO, _D)))
    x = bf.reshape(_NT, _D)

    pad_r = _TILES * _NB * _BS - _E
    r = edge_mat[0].astype(jnp.int32)
    c = edge_mat[1].astype(jnp.int32)
    rows_l = jnp.concatenate(
        [r, _N + (jnp.arange(pad_r, dtype=jnp.int32) % _NDUMP)]
    ).reshape(_TILES, _NB, _BS)
    offs = (jnp.arange(_B, dtype=jnp.int32) * _N)[:, None, None, None]
    c_pad = jnp.concatenate(
        [c, jnp.arange(pad_r, dtype=jnp.int32) % _N]
    ).reshape(_TILES, _NB, _BS)
    extra = jnp.zeros((_TILES, _NBX - _NB, _BS), jnp.int32)
    cols_g = jnp.concatenate([c_pad, extra], axis=1)[None] + offs

    h = x
    reps = [x]
    for layer in range(2):
        p = params["mlp"][layer]
        bn = params["bn"][layer]
        pooled = _sc_segsum(h, cols_g, rows_l)
        z1, st1 = _mm_stats(pooled, p["W1"], p["b1"].reshape(1, _D))
        z2, st2 = _bn_mm_stats(
            z1, st1, p["g1"].reshape(1, _D), p["be1"].reshape(1, _D),
            p["W2"], p["b2"].reshape(1, _D),
        )
        h = _bn_relu(z2, st2, bn["g"].reshape(1, _D), bn["b"].reshape(1, _D))
        reps.append(h)

    em_idx = ((jnp.arange(_B) * _N)[:, None] + jnp.arange(_DO)[None, :]).reshape(-1)
    hs = jnp.stack([hh[em_idx] for hh in reps])
    w = jnp.stack([params["pred"][l]["W"][:, 0] for l in range(3)])
    bsum = (
        params["pred"][0]["b"] + params["pred"][1]["b"] + params["pred"][2]["b"]
    ).reshape(1, 1)
    score = _pred(hs, w, bsum)
    return score.reshape(_B, _DO)


# speedup vs baseline: 1.0593x; 1.0593x over previous
import functools

import jax
import jax.numpy as jnp
from jax import lax
from jax.experimental import pallas as pl
from jax.experimental.pallas import tpu as pltpu
from jax.experimental.pallas import tpu_sc as plsc

_B = 4
_N = 10000
_E = 160000
_D = 128
_DO = 8
_NT = _B * _N
_TILES = 16
_BS = 128
_NB = 80
_CHUNK = 8
_NCHUNK = _NB // _CHUNK
_NBX = _NB + _CHUNK
_RPT = 624
_REM_OFF = _RPT * _TILES
_REM = _N - _REM_OFF
_NDUMP = 256
_ACC_ROWS = _N + _NDUMP
_EPS = 1e-5


def _make_sc_segsum():
    mesh = plsc.VectorSubcoreMesh(core_axis_name="c", subcore_axis_name="s")

    @functools.partial(
        pl.kernel,
        mesh=mesh,
        out_type=jax.ShapeDtypeStruct((_NT, _D), jnp.float32),
        scratch_types=[
            pltpu.VMEM_SHARED((_ACC_ROWS, _D), jnp.float32),
            pltpu.VMEM((2, _CHUNK, _BS), jnp.int32),
            pltpu.VMEM((_NB, _BS), jnp.int32),
            pltpu.VMEM((2, _BS, _D), jnp.float32),
            pltpu.SemaphoreType.DMA((2,)),
            pltpu.SemaphoreType.DMA((2,)),
            pltpu.SemaphoreType.DMA,
        ],
    )
    def segsum(h_hbm, cols_hbm, rows_hbm, out_hbm, acc, idx_c, idx_r, buf,
               gsem, ssem, isem):
        c = lax.axis_index("c")
        s = lax.axis_index("s")
        pltpu.sync_copy(rows_hbm.at[s], idx_r)
        for gi in range(_B // 2):
            b = c + 2 * gi
            pltpu.sync_copy(
                h_hbm.at[pl.ds(b * _N + s * _RPT, _RPT)],
                acc.at[pl.ds(s * _RPT, _RPT)],
            )

            @pl.when(s == 0)
            def _():
                pltpu.sync_copy(
                    h_hbm.at[pl.ds(b * _N + _REM_OFF, _REM)],
                    acc.at[pl.ds(_REM_OFF, _REM)],
                )

            pltpu.sync_copy(cols_hbm.at[b, s, pl.ds(0, _CHUNK)], idx_c.at[0])
            plsc.subcore_barrier()

            def chunk_body(ci, carry):
                cur = ci % 2
                jb = ci * _CHUNK

                def gath_start(k):
                    pltpu.async_copy(h_hbm.at[idx_c.at[cur, k]],
                                     buf.at[k % 2], gsem.at[k % 2])

                def gath_wait(k):
                    pltpu.make_async_copy(h_hbm.at[idx_c.at[cur, k]],
                                          buf.at[k % 2], gsem.at[k % 2]).wait()

                def scat_start(k):
                    pltpu.async_copy(buf.at[k % 2], acc.at[idx_r.at[jb + k]],
                                     ssem.at[k % 2], add=True)

                def scat_wait(k):
                    pltpu.make_async_copy(buf.at[k % 2],
                                          acc.at[idx_r.at[jb + k]],
                                          ssem.at[k % 2]).wait()

                off = pl.multiple_of((ci + 1) * _CHUNK, 8)
                pltpu.async_copy(cols_hbm.at[b, s, pl.ds(off, _CHUNK)],
                                 idx_c.at[(ci + 1) % 2], isem)

                for k in range(_CHUNK):
                    if k >= 2:
                        scat_wait(k - 2)
                    gath_start(k)
                    if k >= 1:
                        gath_wait(k - 1)
                        scat_start(k - 1)
                gath_wait(_CHUNK - 1)
                scat_start(_CHUNK - 1)
                scat_wait(_CHUNK - 2)
                scat_wait(_CHUNK - 1)
                pltpu.make_async_copy(
                    cols_hbm.at[b, s, pl.ds(off, _CHUNK)],
                    idx_c.at[(ci + 1) % 2], isem).wait()
                return carry

            lax.fori_loop(0, _NCHUNK, chunk_body, 0)
            plsc.subcore_barrier()
            pltpu.sync_copy(
                acc.at[pl.ds(s * _RPT, _RPT)],
                out_hbm.at[pl.ds(b * _N + s * _RPT, _RPT)],
            )

            @pl.when(s == 0)
            def _():
                pltpu.sync_copy(
                    acc.at[pl.ds(_REM_OFF, _REM)],
                    out_hbm.at[pl.ds(b * _N + _REM_OFF, _REM)],
                )

            plsc.subcore_barrier()

    return segsum


_sc_segsum = _make_sc_segsum()


_BLK = 4000
_GRID = _NT // _BLK


def _stats_update(i, st_ref, z):
    ps = jnp.concatenate(
        [jnp.sum(z, axis=0, keepdims=True), jnp.sum(z * z, axis=0, keepdims=True)],
        axis=0,
    )

    @pl.when(i == 0)
    def _():
        st_ref[...] = jnp.zeros_like(st_ref)

    st_ref[...] += ps


def _mm_stats_body(x_ref, w_ref, b_ref, z_ref, st_ref):
    i = pl.program_id(0)
    z = jnp.dot(x_ref[...], w_ref[...], preferred_element_type=jnp.float32) + b_ref[...]
    z_ref[...] = z
    _stats_update(i, st_ref, z)


def _mm_stats(x, w, b):
    return pl.pallas_call(
        _mm_stats_body,
        grid=(_GRID,),
        in_specs=[
            pl.BlockSpec((_BLK, _D), lambda i: (i, 0)),
            pl.BlockSpec((_D, _D), lambda i: (0, 0)),
            pl.BlockSpec((1, _D), lambda i: (0, 0)),
        ],
        out_specs=[
            pl.BlockSpec((_BLK, _D), lambda i: (i, 0)),
            pl.BlockSpec((2, _D), lambda i: (0, 0)),
        ],
        out_shape=[
            jax.ShapeDtypeStruct((_NT, _D), jnp.float32),
            jax.ShapeDtypeStruct((2, _D), jnp.float32),
        ],
    )(x, w, b)


def _affine_from_stats(st, g, be):
    mean = st[0:1, :] * (1.0 / _NT)
    var = st[1:2, :] * (1.0 / _NT) - mean * mean
    scale = g * lax.rsqrt(var + _EPS)
    shift = be - mean * scale
    return scale, shift


def _bn_mm_stats_body(z1_ref, st_ref, g_ref, be_ref, w_ref, b_ref, z2_ref, st2_ref):
    i = pl.program_id(0)
    scale, shift = _affine_from_stats(st_ref[...], g_ref[...], be_ref[...])
    x = jnp.maximum(z1_ref[...] * scale + shift, 0.0)
    z2 = jnp.dot(x, w_ref[...], preferred_element_type=jnp.float32) + b_ref[...]
    z2_ref[...] = z2
    _stats_update(i, st2_ref, z2)


def _bn_mm_stats(z1, st, g, be, w, b):
    return pl.pallas_call(
        _bn_mm_stats_body,
        grid=(_GRID,),
        in_specs=[
            pl.BlockSpec((_BLK, _D), lambda i: (i, 0)),
            pl.BlockSpec((2, _D), lambda i: (0, 0)),
            pl.BlockSpec((1, _D), lambda i: (0, 0)),
            pl.BlockSpec((1, _D), lambda i: (0, 0)),
            pl.BlockSpec((_D, _D), lambda i: (0, 0)),
            pl.BlockSpec((1, _D), lambda i: (0, 0)),
        ],
        out_specs=[
            pl.BlockSpec((_BLK, _D), lambda i: (i, 0)),
            pl.BlockSpec((2, _D), lambda i: (0, 0)),
        ],
        out_shape=[
            jax.ShapeDtypeStruct((_NT, _D), jnp.float32),
            jax.ShapeDtypeStruct((2, _D), jnp.float32),
        ],
    )(z1, st, g, be, w, b)


def _bn_relu_body(z_ref, st_ref, g_ref, be_ref, h_ref):
    scale, shift = _affine_from_stats(st_ref[...], g_ref[...], be_ref[...])
    h_ref[...] = jnp.maximum(z_ref[...] * scale + shift, 0.0)


def _bn_relu(z, st, g, be):
    return pl.pallas_call(
        _bn_relu_body,
        grid=(_GRID,),
        in_specs=[
            pl.BlockSpec((_BLK, _D), lambda i: (i, 0)),
            pl.BlockSpec((2, _D), lambda i: (0, 0)),
            pl.BlockSpec((1, _D), lambda i: (0, 0)),
            pl.BlockSpec((1, _D), lambda i: (0, 0)),
        ],
        out_specs=pl.BlockSpec((_BLK, _D), lambda i: (i, 0)),
        out_shape=jax.ShapeDtypeStruct((_NT, _D), jnp.float32),
    )(z, st, g, be)


def _pred_body(hs_ref, w_ref, bsum_ref, out_ref):
    prod = hs_ref[...] * w_ref[...][:, None, :]
    s = jnp.sum(jnp.sum(prod, axis=0), axis=1, keepdims=True)
    out_ref[...] = jax.nn.sigmoid(s + bsum_ref[...])


def _pred(hs, w, bsum):
    return pl.pallas_call(
        _pred_body,
        in_specs=[
            pl.BlockSpec(hs.shape, lambda: (0, 0, 0)),
            pl.BlockSpec(w.shape, lambda: (0, 0)),
            pl.BlockSpec(bsum.shape, lambda: (0, 0)),
        ],
        out_specs=pl.BlockSpec((_B * _DO, 1), lambda: (0, 0)),
        out_shape=jax.ShapeDtypeStruct((_B * _DO, 1), jnp.float32),
    )(hs, w, bsum)


def kernel(batch_feature, edge_mat, params):
    em = params["emotion_embeddings"]
    bf = batch_feature.at[:, :_DO, :].set(jnp.broadcast_to(em, (_B, _DO, _D)))
    x = bf.reshape(_NT, _D)

    pad_r = _TILES * _NB * _BS - _E
    r = edge_mat[0].astype(jnp.int32)
    c = edge_mat[1].astype(jnp.int32)
    rows_l = jnp.concatenate(
        [r, _N + (jnp.arange(pad_r, dtype=jnp.int32) % _NDUMP)]
    ).reshape(_TILES, _NB, _BS)
    offs = (jnp.arange(_B, dtype=jnp.int32) * _N)[:, None, None, None]
    c_pad = jnp.concatenate(
        [c, jnp.arange(pad_r, dtype=jnp.int32) % _N]
    ).reshape(_TILES, _NB, _BS)
    extra = jnp.zeros((_TILES, _NBX - _NB, _BS), jnp.int32)
    cols_g = jnp.concatenate([c_pad, extra], axis=1)[None] + offs

    h = x
    reps = [x]
    for layer in range(2):
        p = params["mlp"][layer]
        bn = params["bn"][layer]
        pooled = _sc_segsum(h, cols_g, rows_l)
        z1, st1 = _mm_stats(pooled, p["W1"], p["b1"].reshape(1, _D))
        z2, st2 = _bn_mm_stats(
            z1, st1, p["g1"].reshape(1, _D), p["be1"].reshape(1, _D),
            p["W2"], p["b2"].reshape(1, _D),
        )
        h = _bn_relu(z2, st2, bn["g"].reshape(1, _D), bn["b"].reshape(1, _D))
        reps.append(h)

    em_idx = ((jnp.arange(_B) * _N)[:, None] + jnp.arange(_DO)[None, :]).reshape(-1)
    hs = jnp.stack([hh[em_idx] for hh in reps])
    w = jnp.stack([params["pred"][l]["W"][:, 0] for l in range(3)])
    bsum = (
        params["pred"][0]["b"] + params["pred"][1]["b"] + params["pred"][2]["b"]
    ).reshape(1, 1)
    score = _pred(hs, w, bsum)
    return score.reshape(_B, _DO)


# speedup vs baseline: 1.0741x; 1.0139x over previous
import functools

import jax
import jax.numpy as jnp
from jax import lax
from jax.experimental import pallas as pl
from jax.experimental.pallas import tpu as pltpu
from jax.experimental.pallas import tpu_sc as plsc

_B = 4
_N = 10000
_E = 160000
_D = 128
_DO = 8
_NT = _B * _N
_TILES = 16
_BS = 128
_NB = 80
_CHUNK = 8
_NCHUNK = _NB // _CHUNK
_NBX = _NB + _CHUNK
_RPT = 624
_REM_OFF = _RPT * _TILES
_REM = _N - _REM_OFF
_NDUMP = 256
_ACC_ROWS = _N + _NDUMP
_EPS = 1e-5


def _make_sc_segsum():
    mesh = plsc.VectorSubcoreMesh(core_axis_name="c", subcore_axis_name="s")

    @functools.partial(
        pl.kernel,
        mesh=mesh,
        out_type=jax.ShapeDtypeStruct((_NT, _D), jnp.float32),
        scratch_types=[
            pltpu.VMEM_SHARED((_ACC_ROWS, _D), jnp.float32),
            pltpu.VMEM((2, _CHUNK, _BS), jnp.int32),
            pltpu.VMEM((_NB, _BS), jnp.int32),
            pltpu.VMEM((2, _BS, _D), jnp.float32),
            pltpu.SemaphoreType.DMA((2,)),
            pltpu.SemaphoreType.DMA((2,)),
            pltpu.SemaphoreType.DMA,
        ],
    )
    def segsum(h_hbm, cols_hbm, rows_hbm, out_hbm, acc, idx_c, idx_r, buf,
               gsem, ssem, isem):
        c = lax.axis_index("c")
        s = lax.axis_index("s")
        pltpu.sync_copy(rows_hbm.at[s], idx_r)
        for gi in range(_B // 2):
            b = c + 2 * gi
            pltpu.sync_copy(
                h_hbm.at[pl.ds(b * _N + s * _RPT, _RPT)],
                acc.at[pl.ds(s * _RPT, _RPT)],
            )

            @pl.when(s == 0)
            def _():
                pltpu.sync_copy(
                    h_hbm.at[pl.ds(b * _N + _REM_OFF, _REM)],
                    acc.at[pl.ds(_REM_OFF, _REM)],
                )

            pltpu.sync_copy(cols_hbm.at[b, s, pl.ds(0, _CHUNK)], idx_c.at[0])
            plsc.subcore_barrier()

            def chunk_body(ci, carry):
                cur = ci % 2
                jb = ci * _CHUNK

                def gath_start(k):
                    pltpu.async_copy(h_hbm.at[idx_c.at[cur, k]],
                                     buf.at[k % 2], gsem.at[k % 2])

                def gath_wait(k):
                    pltpu.make_async_copy(h_hbm.at[idx_c.at[cur, k]],
                                          buf.at[k % 2], gsem.at[k % 2]).wait()

                def scat_start(k):
                    pltpu.async_copy(buf.at[k % 2], acc.at[idx_r.at[jb + k]],
                                     ssem.at[k % 2], add=True)

                def scat_wait(k):
                    pltpu.make_async_copy(buf.at[k % 2],
                                          acc.at[idx_r.at[jb + k]],
                                          ssem.at[k % 2]).wait()

                off = pl.multiple_of((ci + 1) * _CHUNK, 8)
                pltpu.async_copy(cols_hbm.at[b, s, pl.ds(off, _CHUNK)],
                                 idx_c.at[(ci + 1) % 2], isem)

                for k in range(_CHUNK):
                    if k >= 2:
                        scat_wait(k - 2)
                    gath_start(k)
                    if k >= 1:
                        gath_wait(k - 1)
                        scat_start(k - 1)
                gath_wait(_CHUNK - 1)
                scat_start(_CHUNK - 1)
                scat_wait(_CHUNK - 2)
                scat_wait(_CHUNK - 1)
                pltpu.make_async_copy(
                    cols_hbm.at[b, s, pl.ds(off, _CHUNK)],
                    idx_c.at[(ci + 1) % 2], isem).wait()
                return carry

            lax.fori_loop(0, _NCHUNK, chunk_body, 0)
            plsc.subcore_barrier()
            pltpu.sync_copy(
                acc.at[pl.ds(s * _RPT, _RPT)],
                out_hbm.at[pl.ds(b * _N + s * _RPT, _RPT)],
            )

            @pl.when(s == 0)
            def _():
                pltpu.sync_copy(
                    acc.at[pl.ds(_REM_OFF, _REM)],
                    out_hbm.at[pl.ds(b * _N + _REM_OFF, _REM)],
                )

            plsc.subcore_barrier()

    return segsum


_sc_segsum = _make_sc_segsum()


_BLK = 8000
_GRID = _NT // _BLK


def _stats_update(i, st_ref, z):
    ps = jnp.concatenate(
        [jnp.sum(z, axis=0, keepdims=True), jnp.sum(z * z, axis=0, keepdims=True)],
        axis=0,
    )

    @pl.when(i == 0)
    def _():
        st_ref[...] = jnp.zeros_like(st_ref)

    st_ref[...] += ps


def _mm_stats_body(x_ref, w_ref, b_ref, z_ref, st_ref):
    i = pl.program_id(0)
    z = jnp.dot(x_ref[...], w_ref[...], preferred_element_type=jnp.float32) + b_ref[...]
    z_ref[...] = z
    _stats_update(i, st_ref, z)


def _mm_stats(x, w, b):
    return pl.pallas_call(
        _mm_stats_body,
        grid=(_GRID,),
        in_specs=[
            pl.BlockSpec((_BLK, _D), lambda i: (i, 0)),
            pl.BlockSpec((_D, _D), lambda i: (0, 0)),
            pl.BlockSpec((1, _D), lambda i: (0, 0)),
        ],
        out_specs=[
            pl.BlockSpec((_BLK, _D), lambda i: (i, 0)),
            pl.BlockSpec((2, _D), lambda i: (0, 0)),
        ],
        out_shape=[
            jax.ShapeDtypeStruct((_NT, _D), jnp.float32),
            jax.ShapeDtypeStruct((2, _D), jnp.float32),
        ],
    )(x, w, b)


def _affine_from_stats(st, g, be):
    mean = st[0:1, :] * (1.0 / _NT)
    var = st[1:2, :] * (1.0 / _NT) - mean * mean
    scale = g * lax.rsqrt(var + _EPS)
    shift = be - mean * scale
    return scale, shift


def _bn_mm_stats_body(z1_ref, st_ref, g_ref, be_ref, w_ref, b_ref, z2_ref, st2_ref):
    i = pl.program_id(0)
    scale, shift = _affine_from_stats(st_ref[...], g_ref[...], be_ref[...])
    x = jnp.maximum(z1_ref[...] * scale + shift, 0.0)
    z2 = jnp.dot(x, w_ref[...], preferred_element_type=jnp.float32) + b_ref[...]
    z2_ref[...] = z2
    _stats_update(i, st2_ref, z2)


def _bn_mm_stats(z1, st, g, be, w, b):
    return pl.pallas_call(
        _bn_mm_stats_body,
        grid=(_GRID,),
        in_specs=[
            pl.BlockSpec((_BLK, _D), lambda i: (i, 0)),
            pl.BlockSpec((2, _D), lambda i: (0, 0)),
            pl.BlockSpec((1, _D), lambda i: (0, 0)),
            pl.BlockSpec((1, _D), lambda i: (0, 0)),
            pl.BlockSpec((_D, _D), lambda i: (0, 0)),
            pl.BlockSpec((1, _D), lambda i: (0, 0)),
        ],
        out_specs=[
            pl.BlockSpec((_BLK, _D), lambda i: (i, 0)),
            pl.BlockSpec((2, _D), lambda i: (0, 0)),
        ],
        out_shape=[
            jax.ShapeDtypeStruct((_NT, _D), jnp.float32),
            jax.ShapeDtypeStruct((2, _D), jnp.float32),
        ],
    )(z1, st, g, be, w, b)


def _bn_relu_body(z_ref, st_ref, g_ref, be_ref, h_ref):
    scale, shift = _affine_from_stats(st_ref[...], g_ref[...], be_ref[...])
    h_ref[...] = jnp.maximum(z_ref[...] * scale + shift, 0.0)


def _bn_relu(z, st, g, be):
    return pl.pallas_call(
        _bn_relu_body,
        grid=(_GRID,),
        in_specs=[
            pl.BlockSpec((_BLK, _D), lambda i: (i, 0)),
            pl.BlockSpec((2, _D), lambda i: (0, 0)),
            pl.BlockSpec((1, _D), lambda i: (0, 0)),
            pl.BlockSpec((1, _D), lambda i: (0, 0)),
        ],
        out_specs=pl.BlockSpec((_BLK, _D), lambda i: (i, 0)),
        out_shape=jax.ShapeDtypeStruct((_NT, _D), jnp.float32),
    )(z, st, g, be)


def _pred_body(hs_ref, w_ref, bsum_ref, out_ref):
    prod = hs_ref[...] * w_ref[...][:, None, :]
    s = jnp.sum(jnp.sum(prod, axis=0), axis=1, keepdims=True)
    out_ref[...] = jax.nn.sigmoid(s + bsum_ref[...])


def _pred(hs, w, bsum):
    return pl.pallas_call(
        _pred_body,
        in_specs=[
            pl.BlockSpec(hs.shape, lambda: (0, 0, 0)),
            pl.BlockSpec(w.shape, lambda: (0, 0)),
            pl.BlockSpec(bsum.shape, lambda: (0, 0)),
        ],
        out_specs=pl.BlockSpec((_B * _DO, 1), lambda: (0, 0)),
        out_shape=jax.ShapeDtypeStruct((_B * _DO, 1), jnp.float32),
    )(hs, w, bsum)


def kernel(batch_feature, edge_mat, params):
    em = params["emotion_embeddings"]
    bf = batch_feature.at[:, :_DO, :].set(jnp.broadcast_to(em, (_B, _DO, _D)))
    x = bf.reshape(_NT, _D)

    pad_r = _TILES * _NB * _BS - _E
    r = edge_mat[0].astype(jnp.int32)
    c = edge_mat[1].astype(jnp.int32)
    rows_l = jnp.concatenate(
        [r, _N + (jnp.arange(pad_r, dtype=jnp.int32) % _NDUMP)]
    ).reshape(_TILES, _NB, _BS)
    offs = (jnp.arange(_B, dtype=jnp.int32) * _N)[:, None, None, None]
    c_pad = jnp.concatenate(
        [c, jnp.arange(pad_r, dtype=jnp.int32) % _N]
    ).reshape(_TILES, _NB, _BS)
    extra = jnp.zeros((_TILES, _NBX - _NB, _BS), jnp.int32)
    cols_g = jnp.concatenate([c_pad, extra], axis=1)[None] + offs

    h = x
    reps = [x]
    for layer in range(2):
        p = params["mlp"][layer]
        bn = params["bn"][layer]
        pooled = _sc_segsum(h, cols_g, rows_l)
        z1, st1 = _mm_stats(pooled, p["W1"], p["b1"].reshape(1, _D))
        z2, st2 = _bn_mm_stats(
            z1, st1, p["g1"].reshape(1, _D), p["be1"].reshape(1, _D),
            p["W2"], p["b2"].reshape(1, _D),
        )
        h = _bn_relu(z2, st2, bn["g"].reshape(1, _D), bn["b"].reshape(1, _D))
        reps.append(h)

    em_idx = ((jnp.arange(_B) * _N)[:, None] + jnp.arange(_DO)[None, :]).reshape(-1)
    hs = jnp.stack([hh[em_idx] for hh in reps])
    w = jnp.stack([params["pred"][l]["W"][:, 0] for l in range(3)])
    bsum = (
        params["pred"][0]["b"] + params["pred"][1]["b"] + params["pred"][2]["b"]
    ).reshape(1, 1)
    score = _pred(hs, w, bsum)
    return score.reshape(_B, _DO)
